# baseline (speedup 1.0000x reference)
import functools

import jax
import jax.numpy as jnp
from jax import lax
from jax.experimental import pallas as pl
from jax.experimental.pallas import tpu as pltpu
from jax.experimental.pallas import tpu_sc as plsc

F32 = jnp.float32
HI = lax.Precision.HIGHEST
I32 = jnp.int32

N = 10000
B = 512
DF = 64
DE = 16
DIM = 64
M1 = 32
NI2 = 16
D = 128

EP = 20480
NP = 10240
BP = 528
NW = 32
NC = 2
NH = NP // 2
SH = NH + 128


def _mesh():
    return plsc.VectorSubcoreMesh(core_axis_name="c", subcore_axis_name="s")


def _sc_gather(C, K):
    RPW = C * K

    @functools.partial(
        pl.kernel,
        out_type=jax.ShapeDtypeStruct((NW * RPW, D), F32),
        mesh=_mesh(),
        scratch_types=[
            pltpu.VMEM((C, K), I32),
            pltpu.VMEM((RPW, D), F32),
            pltpu.SemaphoreType.DMA,
        ],
    )
    def k(table_hbm, idx_hbm, out_hbm, idx_v, rows_v, sem):
        wid = lax.axis_index("s") * NC + lax.axis_index("c")
        pltpu.sync_copy(idx_hbm.at[wid], idx_v)
        handles = [
            pltpu.async_copy(table_hbm.at[idx_v.at[j]],
                             rows_v.at[pl.ds(j * K, K)], sem)
            for j in range(C)
        ]
        for h in handles:
            h.wait()
        pltpu.sync_copy(rows_v, out_hbm.at[pl.ds(wid * RPW, RPW)])

    return k


def _sc_segsum_big(gather):
    scratch = [
        pltpu.VMEM((10, 128), I32),
        pltpu.VMEM((640, D), F32),
        pltpu.VMEM_SHARED((SH, D), F32),
        pltpu.SemaphoreType.DMA,
    ]
    if gather:
        scratch.insert(1, pltpu.VMEM((10, 128), I32))

    @functools.partial(
        pl.kernel,
        out_type=jax.ShapeDtypeStruct((NC, SH, D), F32),
        mesh=_mesh(),
        scratch_types=scratch,
    )
    def k(vals_hbm, dst_hbm, *rest):
        if gather:
            src_hbm, zeros_hbm, out_hbm, dst_v, src_v, rows_v, accum, sem = rest
        else:
            zeros_hbm, out_hbm, dst_v, rows_v, accum, sem = rest
        cid = lax.axis_index("c")
        sid = lax.axis_index("s")
        pltpu.sync_copy(dst_hbm.at[cid * 16 + sid], dst_v)
        if gather:
            pltpu.sync_copy(src_hbm.at[sid], src_v)

        @pl.when(sid == 0)
        def _():
            pltpu.sync_copy(zeros_hbm, accum)

        plsc.subcore_barrier()
        for r in range(2):
            if gather:
                handles = [
                    pltpu.async_copy(vals_hbm.at[src_v.at[5 * r + j]],
                                     rows_v.at[pl.ds(j * 128, 128)], sem)
                    for j in range(5)
                ]
            else:
                handles = [pltpu.async_copy(
                    vals_hbm.at[pl.ds(sid * 1280 + r * 640, 640)],
                    rows_v, sem)]
            for h in handles:
                h.wait()
            for j in range(5):
                pltpu.sync_copy(rows_v.at[pl.ds(j * 128, 128)],
                                accum.at[dst_v.at[5 * r + j]], add=True)
        plsc.subcore_barrier()

        @pl.when(sid == 0)
        def _():
            pltpu.sync_copy(accum, out_hbm.at[cid])

    return k


def _sc_segsum_small(S, C, K, gather):
    RPW = C * K
    scratch = [
        pltpu.VMEM((C, K), I32),
        pltpu.VMEM((RPW, D), F32),
        pltpu.VMEM_SHARED((S, D), F32),
        pltpu.SemaphoreType.DMA,
    ]
    if gather:
        scratch.insert(1, pltpu.VMEM((C, K), I32))

    @functools.partial(
        pl.kernel,
        out_type=jax.ShapeDtypeStruct((NC, S, D), F32),
        mesh=_mesh(),
        scratch_types=scratch,
    )
    def k(vals_hbm, dst_hbm, *rest):
        if gather:
            src_hbm, zeros_hbm, out_hbm, dst_v, src_v, rows_v, accum, sem = rest
        else:
            zeros_hbm, out_hbm, dst_v, rows_v, accum, sem = rest
        cid = lax.axis_index("c")
        sid = lax.axis_index("s")
        wid = sid * NC + cid
        pltpu.sync_copy(dst_hbm.at[wid], dst_v)
        if gather:
            pltpu.sync_copy(src_hbm.at[wid], src_v)
            handles = [
                pltpu.async_copy(vals_hbm.at[src_v.at[j]],
                                 rows_v.at[pl.ds(j * K, K)], sem)
                for j in range(C)
            ]
        else:
            handles = [pltpu.async_copy(
                vals_hbm.at[pl.ds(wid * RPW, RPW)], rows_v, sem)]

        @pl.when(sid == 0)
        def _():
            pltpu.sync_copy(zeros_hbm, accum)

        plsc.subcore_barrier()
        for h in handles:
            h.wait()
        for j in range(C):
            pltpu.sync_copy(rows_v.at[pl.ds(j * K, K)],
                            accum.at[dst_v.at[j]], add=True)
        plsc.subcore_barrier()

        @pl.when(sid == 0)
        def _():
            pltpu.sync_copy(accum, out_hbm.at[cid])

    return k


def _msg_body(Min, Mout, ea_ref, h_ref, Wa_ref, ba_ref, Wb_ref, bb_ref, o_ref):
    e = jnp.maximum(
        jnp.dot(ea_ref[...], Wa_ref[...], preferred_element_type=F32, precision=HI)
        + ba_ref[...], 0.0)
    w = jnp.dot(e, Wb_ref[...], preferred_element_type=F32, precision=HI) + bb_ref[...]
    hr = jnp.repeat(h_ref[:, :Min], Mout, axis=1)
    p = hr * w
    width = Min * Mout
    while width > Mout:
        width //= 2
        p = p[:, :width] + p[:, width:]
    o_ref[...] = jnp.pad(p, ((0, 0), (0, D - Mout)))


def _msg_call(Min, Mout, TE=512):
    body = functools.partial(_msg_body, Min, Mout)
    return pl.pallas_call(
        body,
        grid=(EP // TE,),
        in_specs=[
            pl.BlockSpec((TE, DE), lambda i: (i, 0)),
            pl.BlockSpec((TE, D), lambda i: (i, 0)),
            pl.BlockSpec((DE, 128), lambda i: (0, 0)),
            pl.BlockSpec((1, 128), lambda i: (0, 0)),
            pl.BlockSpec((128, Min * Mout), lambda i: (0, 0)),
            pl.BlockSpec((1, Min * Mout), lambda i: (0, 0)),
        ],
        out_specs=pl.BlockSpec((TE, D), lambda i: (i, 0)),
        out_shape=jax.ShapeDtypeStruct((EP, D), F32),
    )


def _node_call(body, n_parts, n_node, wspecs, TN=512):
    tpc = NH // TN
    in_specs = ([pl.BlockSpec((1, TN, D), lambda i: (i // tpc, i % tpc, 0))]
                * n_parts
                + [pl.BlockSpec((TN, D), lambda i: (i, 0))] * n_node
                + [pl.BlockSpec(s, lambda i: (0, 0)) for s in wspecs])
    return pl.pallas_call(
        body,
        grid=(NP // TN,),
        in_specs=in_specs,
        out_specs=pl.BlockSpec((TN, D), lambda i: (i, 0)),
        out_shape=jax.ShapeDtypeStruct((NP, D), F32),
    )


def _zpad(v, rows):
    return jnp.pad(v, ((0, 0), (0, D - v.shape[1])))


def _h1_body(aggp_ref, x_ref, Wr_ref, b_ref, o_ref):
    agg = aggp_ref[0, :, :M1]
    h1 = jnp.maximum(
        agg + jnp.dot(x_ref[:, :DF], Wr_ref[...], preferred_element_type=F32, precision=HI)
        + b_ref[...], 0.0)
    o_ref[...] = jnp.pad(h1, ((0, 0), (0, D - M1)))


def _h2_body(aggp_ref, h_ref, Wr_ref, b_ref, o_ref):
    agg = aggp_ref[0, :, :DIM]
    h2 = jnp.maximum(
        agg + jnp.dot(h_ref[:, :M1], Wr_ref[...], preferred_element_type=F32, precision=HI)
        + b_ref[...], 0.0)
    o_ref[...] = jnp.concatenate(
        [h2, jnp.ones((h2.shape[0], NI2), F32),
         jnp.zeros((h2.shape[0], D - DIM - NI2), F32)], axis=1)


def _pool_body(sump_ref, iso_ref, o_ref):
    s = sump_ref[0]
    cnt = jnp.maximum(s[:, DIM:DIM + 1], 1.0)
    o_ref[...] = jnp.concatenate(
        [s[:, :DIM] / cnt, iso_ref[:, :NI2],
         jnp.zeros((s.shape[0], D - DIM - NI2), F32)], axis=1)


def _gconv_body(Din, aggp_ref, h_ref, Wrel_ref, brel_ref, Wroot_ref, o_ref):
    agg = aggp_ref[0, :, :Din]
    g = jnp.maximum(
        jnp.dot(agg, Wrel_ref[...], preferred_element_type=F32, precision=HI)
        + brel_ref[...]
        + jnp.dot(h_ref[:, :Din], Wroot_ref[...],
                  preferred_element_type=F32, precision=HI), 0.0)
    o_ref[...] = jnp.pad(g, ((0, 0), (0, D - DIM)))


def _head_body(x1p_ref, x2p_ref, Wo1_ref, bo1_ref, Wo2_ref, bo2_ref,
               Wo3_ref, bo3_ref, o_ref):
    x1 = (x1p_ref[0] + x1p_ref[1])[:B, :DIM]
    x2 = (x2p_ref[0] + x2p_ref[1])[:B, :DIM]
    m = jnp.concatenate([x1, x2], axis=1)
    o = jnp.maximum(
        jnp.dot(m, Wo1_ref[...], preferred_element_type=F32, precision=HI) + bo1_ref[...],
        0.0)
    o = jnp.maximum(
        jnp.dot(o, Wo2_ref[...], preferred_element_type=F32, precision=HI) + bo2_ref[...],
        0.0)
    o_ref[...] = (jnp.dot(o, Wo3_ref[...], preferred_element_type=F32, precision=HI)
                  + bo3_ref[...])


def _pad_tab(a, rows):
    return jnp.pad(a, ((0, rows - a.shape[0]), (0, D - a.shape[1])))


def _pad_idx(idx, rows, fill, C, K):
    idx = idx.astype(I32)
    idx = jnp.pad(idx, (0, rows - idx.shape[0]), constant_values=fill)
    return idx.reshape(NW, C, K)


def _split_dst(idx):
    p = jnp.pad(idx.astype(I32), (0, EP - idx.shape[0]), constant_values=N)
    tr = NH + (jnp.arange(EP, dtype=I32) % 128)
    d0 = jnp.where(p < NH, p, tr)
    d1 = jnp.where(p >= NH, p - NH, tr)
    return jnp.stack([d0, d1]).reshape(NC * 16, 10, 128)


def _src_big(idx):
    p = jnp.pad(idx.astype(I32), (0, EP - idx.shape[0]), constant_values=0)
    return p.reshape(16, 10, 128)


def kernel(x, edge_index, edge_attr, batch, assignment_index_2, iso_type_2,
           edge_index_2, batch_2, W_e1a, b_e1a, W_e1b, b_e1b, W_r1, b1,
           W_e2a, b_e2a, W_e2b, b_e2b, W_r2, b2, W4_rel, b4, W4_root,
           W5_rel, b5, W5_root, Wo1, bo1, Wo2, bo2, Wo3, bo3):
    r1 = lambda v: v.reshape(1, -1)

    x_p = _pad_tab(x, NP)
    ea_p = jnp.pad(edge_attr, ((0, EP - edge_attr.shape[0]), (0, 0)))
    iso_p = _pad_tab(iso_type_2, NP)

    src = _pad_idx(edge_index[0], EP, 0, 5, 128)
    dst = _split_dst(edge_index[1])
    sa = _src_big(assignment_index_2[0])
    da = _split_dst(assignment_index_2[1])
    s2 = _src_big(edge_index_2[0])
    d2 = _split_dst(edge_index_2[1])
    bt = _pad_idx(batch, NP, B, 4, 80)
    bt2 = _pad_idx(batch_2, NP, B, 4, 80)

    zn = jnp.zeros((SH, D), F32)
    zb = jnp.zeros((BP, D), F32)

    gth_e = _sc_gather(5, 128)
    seg_e = _sc_segsum_big(False)
    seg_eg = _sc_segsum_big(True)
    seg_b = _sc_segsum_small(BP, 4, 80, False)

    hsrc1 = gth_e(x_p, src)
    msg1 = _msg_call(DF, M1)(ea_p, hsrc1, W_e1a, r1(b_e1a), W_e1b, r1(b_e1b))
    agg1p = seg_e(msg1, dst, zn)
    h1 = _node_call(_h1_body, 1, 1, [(DF, M1), (1, M1)])(
        agg1p, x_p, W_r1, r1(b1))
    hsrc2 = gth_e(h1, src)
    msg2 = _msg_call(M1, DIM)(ea_p, hsrc2, W_e2a, r1(b_e2a), W_e2b, r1(b_e2b))
    agg2p = seg_e(msg2, dst, zn)
    h2t = _node_call(_h2_body, 1, 1, [(M1, DIM), (1, DIM)])(
        agg2p, h1, W_r2, r1(b2))

    x1p = seg_b(h2t, bt, zb)

    poolp = seg_eg(h2t, da, sa, zn)
    h2c = _node_call(_pool_body, 1, 1, [])(poolp, iso_p)

    g1p = seg_eg(h2c, d2, s2, zn)
    h3 = _node_call(functools.partial(_gconv_body, DIM + NI2), 1, 1,
                    [(DIM + NI2, DIM), (1, DIM), (DIM + NI2, DIM)])(
        g1p, h2c, W4_rel, r1(b4), W4_root)
    g2p = seg_eg(h3, d2, s2, zn)
    h4 = _node_call(functools.partial(_gconv_body, DIM), 1, 1,
                    [(DIM, DIM), (1, DIM), (DIM, DIM)])(
        g2p, h3, W5_rel, r1(b5), W5_root)

    x2p = seg_b(h4, bt2, zb)

    o = pl.pallas_call(
        _head_body,
        out_shape=jax.ShapeDtypeStruct((B, 1), F32),
    )(x1p, x2p, Wo1, r1(bo1), Wo2, r1(bo2), Wo3, r1(bo3))
    return o.reshape(-1)


# speedup vs baseline: 1.1345x; 1.1345x over previous
import functools

import jax
import jax.numpy as jnp
from jax import lax
from jax.experimental import pallas as pl
from jax.experimental.pallas import tpu as pltpu
from jax.experimental.pallas import tpu_sc as plsc

F32 = jnp.float32
I32 = jnp.int32

N = 10000
B = 512
DF = 64
DE = 16
DIM = 64
M1 = 32
NI2 = 16
D = 128

EP = 20480
NP = 10240
BP = 528
NW = 32
NC = 2
TN = 512
NH = NP // 2
SH = NH + 512
NHB = NH // TN
SHB = SH // TN


def _mesh():
    return plsc.VectorSubcoreMesh(core_axis_name="c", subcore_axis_name="s")


def _sc_gather(C, K):
    RPW = C * K

    @functools.partial(
        pl.kernel,
        out_type=jax.ShapeDtypeStruct((NW * RPW, D), F32),
        mesh=_mesh(),
        scratch_types=[
            pltpu.VMEM((C, K), I32),
            pltpu.VMEM((RPW, D), F32),
            pltpu.SemaphoreType.DMA,
        ],
    )
    def k(table_hbm, idx_hbm, out_hbm, idx_v, rows_v, sem):
        wid = lax.axis_index("s") * NC + lax.axis_index("c")
        pltpu.sync_copy(idx_hbm.at[wid], idx_v)
        handles = [
            pltpu.async_copy(table_hbm.at[idx_v.at[j]],
                             rows_v.at[pl.ds(j * K, K)], sem)
            for j in range(C)
        ]
        for h in handles:
            h.wait()
        pltpu.sync_copy(rows_v, out_hbm.at[pl.ds(wid * RPW, RPW)])

    return k


def _sc_segsum_b(S, C, K):
    RPW = C * K

    @functools.partial(
        pl.kernel,
        out_type=jax.ShapeDtypeStruct((NC * S, D), F32),
        mesh=_mesh(),
        scratch_types=[
            pltpu.VMEM((C, K), I32),
            pltpu.VMEM((RPW, D), F32),
            pltpu.VMEM_SHARED((S, D), F32),
            pltpu.SemaphoreType.DMA,
        ],
    )
    def k(vals_hbm, dst_hbm, zeros_hbm, out_hbm, dst_v, rows_v, accum, sem):
        cid = lax.axis_index("c")
        sid = lax.axis_index("s")
        wid = sid * NC + cid
        pltpu.sync_copy(dst_hbm.at[wid], dst_v)
        h = pltpu.async_copy(vals_hbm.at[pl.ds(wid * RPW, RPW)], rows_v, sem)

        @pl.when(sid == 0)
        def _():
            pltpu.sync_copy(zeros_hbm, accum)

        plsc.subcore_barrier()
        h.wait()
        for j in range(C):
            pltpu.sync_copy(rows_v.at[pl.ds(j * K, K)],
                            accum.at[dst_v.at[j]], add=True)
        plsc.subcore_barrier()

        @pl.when(sid == 0)
        def _():
            pltpu.sync_copy(accum, out_hbm.at[pl.ds(cid * S, S)])

    return k


def _sc_segsum(C, K, gather):
    RPW = C * K
    rps = SH // 16
    scratch = [
        pltpu.VMEM((2 * C, K), I32),
        pltpu.VMEM((RPW, D), F32),
        pltpu.VMEM_SHARED((SH, D), F32),
        pltpu.SemaphoreType.DMA,
    ]
    if gather:
        scratch.insert(1, pltpu.VMEM((C, K), I32))

    @functools.partial(
        pl.kernel,
        out_type=jax.ShapeDtypeStruct((NC * 2 * SH, D), F32),
        mesh=_mesh(),
        scratch_types=scratch,
    )
    def k(vals_hbm, dst_hbm, *rest):
        if gather:
            src_hbm, zeros_hbm, out_hbm, dst_v, src_v, rows_v, acc, sem = rest
        else:
            zeros_hbm, out_hbm, dst_v, rows_v, acc, sem = rest
        cid = lax.axis_index("c")
        sid = lax.axis_index("s")
        wid = sid * NC + cid
        pltpu.sync_copy(dst_hbm.at[wid], dst_v)
        if gather:
            pltpu.sync_copy(src_hbm.at[wid], src_v)
            handles = [
                pltpu.async_copy(vals_hbm.at[src_v.at[j]],
                                 rows_v.at[pl.ds(j * K, K)], sem)
                for j in range(C)
            ]
        else:
            handles = [pltpu.async_copy(
                vals_hbm.at[pl.ds(wid * RPW, RPW)], rows_v, sem)]

        pltpu.sync_copy(zeros_hbm, acc.at[pl.ds(sid * rps, rps)])
        plsc.subcore_barrier()
        for h in handles:
            h.wait()
        for p in range(2):
            for j in range(C):
                pltpu.sync_copy(rows_v.at[pl.ds(j * K, K)],
                                acc.at[dst_v.at[p * C + j]], add=True)
            plsc.subcore_barrier()

            @pl.when(sid == 0)
            def _():
                pltpu.sync_copy(acc, out_hbm.at[pl.ds((cid * 2 + p) * SH,
                                                      SH)])
            if p == 0:
                plsc.subcore_barrier()
                pltpu.sync_copy(zeros_hbm, acc.at[pl.ds(sid * rps, rps)])
                plsc.subcore_barrier()

    return k


def _msg_body(Min, Mout, ea_ref, h_ref, Wa_ref, ba_ref, Wb_ref, bb_ref, o_ref):
    e = jnp.maximum(
        jnp.dot(ea_ref[...], Wa_ref[...], preferred_element_type=F32) + ba_ref[...], 0.0)
    w = jnp.dot(e, Wb_ref[...], preferred_element_type=F32) + bb_ref[...]
    hr = jnp.repeat(h_ref[:, :Min], Mout, axis=1)
    hr = hr.astype(jnp.bfloat16).astype(F32)
    wb = w.astype(jnp.bfloat16).astype(F32)
    p = hr * wb
    width = Min * Mout
    while width > Mout:
        width //= 2
        p = p[:, :width] + p[:, width:]
    o_ref[...] = jnp.pad(p, ((0, 0), (0, D - Mout)))


def _msg_call(Min, Mout, TE=512):
    body = functools.partial(_msg_body, Min, Mout)
    return pl.pallas_call(
        body,
        grid=(EP // TE,),
        in_specs=[
            pl.BlockSpec((TE, DE), lambda i: (i, 0)),
            pl.BlockSpec((TE, D), lambda i: (i, 0)),
            pl.BlockSpec((DE, 128), lambda i: (0, 0)),
            pl.BlockSpec((1, 128), lambda i: (0, 0)),
            pl.BlockSpec((128, Min * Mout), lambda i: (0, 0)),
            pl.BlockSpec((1, Min * Mout), lambda i: (0, 0)),
        ],
        out_specs=pl.BlockSpec((TE, D), lambda i: (i, 0)),
        out_shape=jax.ShapeDtypeStruct((EP, D), F32),
    )


def _node_call(body, n_parts, n_node, wspecs):
    in_specs = []
    for _ in range(n_parts):
        in_specs.append(pl.BlockSpec(
            (TN, D), lambda i: ((i // NHB) * SHB + i % NHB, 0)))
        in_specs.append(pl.BlockSpec(
            (TN, D), lambda i: ((2 + i // NHB) * SHB + i % NHB, 0)))
    in_specs += [pl.BlockSpec((TN, D), lambda i: (i, 0))] * n_node
    in_specs += [pl.BlockSpec(s, lambda i: (0, 0)) for s in wspecs]
    call = pl.pallas_call(
        body,
        grid=(NP // TN,),
        in_specs=in_specs,
        out_specs=pl.BlockSpec((TN, D), lambda i: (i, 0)),
        out_shape=jax.ShapeDtypeStruct((NP, D), F32),
    )

    def run(*args):
        dup = []
        for p in args[:n_parts]:
            dup += [p, p]
        return call(*dup, *args[n_parts:])

    return run


def _h1_body(a0_ref, a1_ref, x_ref, Wr_ref, b_ref, o_ref):
    agg = (a0_ref[...] + a1_ref[...])[:, :M1]
    h1 = jnp.maximum(
        agg + jnp.dot(x_ref[:, :DF], Wr_ref[...], preferred_element_type=F32,
                      ) + b_ref[...], 0.0)
    o_ref[...] = jnp.pad(h1, ((0, 0), (0, D - M1)))


def _h2_body(a0_ref, a1_ref, h_ref, Wr_ref, b_ref, o_ref):
    agg = (a0_ref[...] + a1_ref[...])[:, :DIM]
    h2 = jnp.maximum(
        agg + jnp.dot(h_ref[:, :M1], Wr_ref[...], preferred_element_type=F32,
                      ) + b_ref[...], 0.0)
    o_ref[...] = jnp.concatenate(
        [h2, jnp.ones((h2.shape[0], NI2), F32),
         jnp.zeros((h2.shape[0], D - DIM - NI2), F32)], axis=1)


def _pool_body(s0_ref, s1_ref, iso_ref, o_ref):
    s = s0_ref[...] + s1_ref[...]
    cnt = jnp.maximum(s[:, DIM:DIM + 1], 1.0)
    o_ref[...] = jnp.concatenate(
        [s[:, :DIM] / cnt, iso_ref[:, :NI2],
         jnp.zeros((s.shape[0], D - DIM - NI2), F32)], axis=1)


def _gconv_body(Din, a0_ref, a1_ref, h_ref, Wrel_ref, brel_ref, Wroot_ref,
                o_ref):
    agg = (a0_ref[...] + a1_ref[...])[:, :Din]
    g = jnp.maximum(
        jnp.dot(agg, Wrel_ref[...], preferred_element_type=F32)
        + brel_ref[...]
        + jnp.dot(h_ref[:, :Din], Wroot_ref[...],
                  preferred_element_type=F32), 0.0)
    o_ref[...] = jnp.pad(g, ((0, 0), (0, D - DIM)))


def _head_body(x1p_ref, x2p_ref, Wo1_ref, bo1_ref, Wo2_ref, bo2_ref,
               Wo3_ref, bo3_ref, o_ref):
    x1 = (x1p_ref[:B] + x1p_ref[BP:BP + B])[:, :DIM]
    x2 = (x2p_ref[:B] + x2p_ref[BP:BP + B])[:, :DIM]
    m = jnp.concatenate([x1, x2], axis=1)
    o = jnp.maximum(
        jnp.dot(m, Wo1_ref[...], preferred_element_type=F32)
        + bo1_ref[...], 0.0)
    o = jnp.maximum(
        jnp.dot(o, Wo2_ref[...], preferred_element_type=F32)
        + bo2_ref[...], 0.0)
    o_ref[...] = (jnp.dot(o, Wo3_ref[...], preferred_element_type=F32,
                          ) + bo3_ref[...])


def _pad_tab(a, rows):
    return jnp.pad(a, ((0, rows - a.shape[0]), (0, D - a.shape[1])))


def _pad_idx(idx, rows, fill, C, K):
    idx = idx.astype(I32)
    idx = jnp.pad(idx, (0, rows - idx.shape[0]), constant_values=fill)
    return idx.reshape(NW, C, K)


def _dst2_idx(idx, rows, C, K):
    p = jnp.pad(idx.astype(I32), (0, rows - idx.shape[0]),
                constant_values=NP)
    tr = NH + (jnp.arange(rows, dtype=I32) % (SH - NH))
    da = jnp.where(p < NH, p, tr).reshape(NW, C, K)
    db = jnp.where(p >= NH, p - NH, tr).reshape(NW, C, K)
    return jnp.concatenate([da, db], axis=1)


def kernel(x, edge_index, edge_attr, batch, assignment_index_2, iso_type_2,
           edge_index_2, batch_2, W_e1a, b_e1a, W_e1b, b_e1b, W_r1, b1,
           W_e2a, b_e2a, W_e2b, b_e2b, W_r2, b2, W4_rel, b4, W4_root,
           W5_rel, b5, W5_root, Wo1, bo1, Wo2, bo2, Wo3, bo3):
    r1 = lambda v: v.reshape(1, -1)

    x_p = _pad_tab(x, NP)
    ea_p = jnp.pad(edge_attr, ((0, EP - edge_attr.shape[0]), (0, 0)))
    iso_p = _pad_tab(iso_type_2, NP)

    src = _pad_idx(edge_index[0], EP, 0, 5, 128)
    dst = _dst2_idx(edge_index[1], EP, 5, 128)
    sa = _pad_idx(assignment_index_2[0], EP, 0, 5, 128)
    da = _dst2_idx(assignment_index_2[1], EP, 5, 128)
    s2 = _pad_idx(edge_index_2[0], EP, 0, 5, 128)
    d2 = _dst2_idx(edge_index_2[1], EP, 5, 128)
    bt = _pad_idx(batch, NP, B, 4, 80)
    bt2 = _pad_idx(batch_2, NP, B, 4, 80)

    zn = jnp.zeros((SH // 16, D), F32)
    zb = jnp.zeros((BP, D), F32)

    gth_e = _sc_gather(5, 128)
    seg_e = _sc_segsum(5, 128, False)
    seg_eg = _sc_segsum(5, 128, True)
    seg_b = _sc_segsum_b(BP, 4, 80)

    hsrc1 = gth_e(x_p, src)
    msg1 = _msg_call(DF, M1)(ea_p, hsrc1, W_e1a, r1(b_e1a), W_e1b, r1(b_e1b))
    agg1p = seg_e(msg1, dst, zn)
    h1 = _node_call(_h1_body, 1, 1, [(DF, M1), (1, M1)])(
        agg1p, x_p, W_r1, r1(b1))
    hsrc2 = gth_e(h1, src)
    msg2 = _msg_call(M1, DIM)(ea_p, hsrc2, W_e2a, r1(b_e2a), W_e2b, r1(b_e2b))
    agg2p = seg_e(msg2, dst, zn)
    h2t = _node_call(_h2_body, 1, 1, [(M1, DIM), (1, DIM)])(
        agg2p, h1, W_r2, r1(b2))

    x1p = seg_b(h2t, bt, zb)

    poolp = seg_eg(h2t, da, sa, zn)
    h2c = _node_call(_pool_body, 1, 1, [])(poolp, iso_p)

    g1p = seg_eg(h2c, d2, s2, zn)
    h3 = _node_call(functools.partial(_gconv_body, DIM + NI2), 1, 1,
                    [(DIM + NI2, DIM), (1, DIM), (DIM + NI2, DIM)])(
        g1p, h2c, W4_rel, r1(b4), W4_root)
    g2p = seg_eg(h3, d2, s2, zn)
    h4 = _node_call(functools.partial(_gconv_body, DIM), 1, 1,
                    [(DIM, DIM), (1, DIM), (DIM, DIM)])(
        g2p, h3, W5_rel, r1(b5), W5_root)

    x2p = seg_b(h4, bt2, zb)

    o = pl.pallas_call(
        _head_body,
        out_shape=jax.ShapeDtypeStruct((B, 1), F32),
    )(x1p, x2p, Wo1, r1(bo1), Wo2, r1(bo2), Wo3, r1(bo3))
    return o.reshape(-1)
